# baseline (speedup 1.0000x reference)
import functools

import jax
import jax.numpy as jnp
from jax import lax
from jax.experimental import pallas as pl
from jax.experimental.pallas import tpu as pltpu
from jax.experimental.pallas import tpu_sc as plsc

_B, _C, _D = 16384, 26, 128
_NC, _NS = 2, 16
_NW = _NC * _NS
_RPW = _B // _NW
_NG = _RPW // 16
_H = _RPW // 2
_NU = 2 * _C
_CELL = _H + 16
_TRASH = _H + 32
_IDXN = _H + 48


def _sc_body(x_hbm, mask_hbm, out_hbm,
             mask_v, idxlo, idxhi, bufa, bufb, bufc,
             gsema, gsemb, gsemc, ssema, ssemb, ssemc, msem):
    wid = lax.axis_index("s") * _NC + lax.axis_index("c")
    base = wid * _RPW

    pltpu.make_async_copy(mask_hbm.at[pl.ds(base, _RPW)], mask_v, msem).start()
    pltpu.make_async_copy(mask_hbm.at[pl.ds(base, _RPW)], mask_v, msem).wait()

    lanes = lax.iota(jnp.int32, 16)
    dnums = lax.GatherDimensionNumbers(
        offset_dims=(), collapsed_slice_dims=(0,), start_index_map=(0,))

    def _lane_gather(v, idx):
        return lax.gather(v, idx[:, None], dnums, slice_sizes=(1,),
                          mode=lax.GatherScatterMode.PROMISE_IN_BOUNDS)

    def _incl_cumsum(v):
        s = v
        for sh in (1, 2, 4, 8):
            g = _lane_gather(s, jnp.maximum(lanes - sh, 0))
            s = jnp.where(lanes >= sh, s + g, s)
        return s

    trash = lanes + _TRASH
    nf = [jnp.int32(0), jnp.int32(0)]
    lists = [idxlo, idxhi]
    for g in range(_NG):
        half = g // (_NG // 2)
        dst = lists[half]
        mv = mask_v[pl.ds(g * 16, 16)]
        rows = lanes + (g % (_NG // 2)) * 16
        m_i = jnp.where(mv != 0, 1, 0)
        incl = _incl_cumsum(m_i)
        pos_f = jnp.where(mv != 0, trash, nf[half] + lanes - incl)
        plsc.store_scatter(dst, [pos_f], rows)
        plsc.store_scatter(dst, [jnp.where(mv != 0, trash, _CELL)], rows)
        nf[half] = nf[half] + (16 - incl[15])
    for half in range(2):
        cell = jnp.full((16,), lists[half][pl.ds(_CELL, 16)][0], jnp.int32)
        lists[half][pl.ds(nf[half], 16)] = cell
    ngrp = [(nf[0] + 15) >> 4, (nf[1] + 15) >> 4]

    bufs = [bufa, bufb, bufc]
    gsems = [gsema, gsemb, gsemc]
    ssems = [ssema, ssemb, ssemc]
    z16 = jnp.zeros((16,), jnp.float32)

    def _unit_slice(u):
        c = u >> 1
        h = u & 1
        return pl.ds((c * _B + base + h * _H) * _D, _H * _D)

    def _gather(u, r):
        return pltpu.make_async_copy(
            x_hbm.at[_unit_slice(u)], bufs[r], gsems[r])

    def _scatter(u, r):
        return pltpu.make_async_copy(
            bufs[r], out_hbm.at[_unit_slice(u)], ssems[r])

    def _zero_false(half, r):
        buf = bufs[r]

        def body(g, carry):
            v = lists[half][pl.ds(g * 16, 16)]
            pv = v * _D
            for j in range(_D):
                plsc.store_scatter(buf, [pv + j], z16)
            return carry

        lax.fori_loop(0, ngrp[half], body, jnp.int32(0))

    _gather(0, 0).start()

    def qbody(q, carry):
        u0 = 6 * q
        for j in range(6):
            u = u0 + j
            r = j % 3
            rn = (j + 1) % 3

            @pl.when(u < _NU)
            def _():
                _gather(u, r).wait()
                _zero_false(j & 1, r)
                _scatter(u, r).start()

            un = u + 1

            @pl.when((un < _NU) & (un >= 3))
            def _():
                _scatter(un - 3, rn).wait()

            @pl.when(un < _NU)
            def _():
                _gather(un, rn).start()

        return carry

    lax.fori_loop(0, (_NU + 5) // 6, qbody, jnp.int32(0))
    for r in range(3):
        _scatter(0, r).wait()


_sc_call = functools.partial(
    pl.kernel,
    out_type=jax.ShapeDtypeStruct((_C * _B * _D,), jnp.float32),
    mesh=plsc.VectorSubcoreMesh(core_axis_name="c", subcore_axis_name="s"),
    compiler_params=pltpu.CompilerParams(needs_layout_passes=False),
    scratch_types=[
        pltpu.VMEM((_RPW,), jnp.int32),
        pltpu.VMEM((_IDXN,), jnp.int32),
        pltpu.VMEM((_IDXN,), jnp.int32),
        pltpu.VMEM((_H * _D,), jnp.float32),
        pltpu.VMEM((_H * _D,), jnp.float32),
        pltpu.VMEM((_H * _D,), jnp.float32),
        pltpu.SemaphoreType.DMA,
        pltpu.SemaphoreType.DMA,
        pltpu.SemaphoreType.DMA,
        pltpu.SemaphoreType.DMA,
        pltpu.SemaphoreType.DMA,
        pltpu.SemaphoreType.DMA,
        pltpu.SemaphoreType.DMA,
    ],
)(_sc_body)


def kernel(x, mask):
    x2 = jnp.transpose(x, (1, 0, 2)).reshape(_C * _B * _D)
    mask_i32 = mask.astype(jnp.int32)
    y2 = _sc_call(x2, mask_i32)
    return jnp.transpose(y2.reshape(_C, _B, _D), (1, 0, 2))


# speedup vs baseline: 2.6712x; 2.6712x over previous
import functools

import jax
import jax.numpy as jnp
from jax import lax
from jax.experimental import pallas as pl
from jax.experimental.pallas import tpu as pltpu
from jax.experimental.pallas import tpu_sc as plsc

_B, _C, _D = 16384, 26, 128
_NC, _NS = 2, 16
_NW = _NC * _NS
_RPW = _B // _NW
_NG = _RPW // 16
_H = _RPW // 2
_NU = 2 * _C
_CELL = _H + 16
_TRASH = _H + 32
_IDXN = _H + 48


def _sc_body(x_hbm, mask_hbm, out_hbm,
             mask_v, idxlo, idxhi, bufa, bufb, bufc,
             gsema, gsemb, gsemc, ssema, ssemb, ssemc, msem):
    wid = lax.axis_index("s") * _NC + lax.axis_index("c")
    base = wid * _RPW

    pltpu.make_async_copy(mask_hbm.at[pl.ds(base, _RPW)], mask_v, msem).start()
    pltpu.make_async_copy(mask_hbm.at[pl.ds(base, _RPW)], mask_v, msem).wait()

    lanes = lax.iota(jnp.int32, 16)
    dnums = lax.GatherDimensionNumbers(
        offset_dims=(), collapsed_slice_dims=(0,), start_index_map=(0,))

    def _lane_gather(v, idx):
        return lax.gather(v, idx[:, None], dnums, slice_sizes=(1,),
                          mode=lax.GatherScatterMode.PROMISE_IN_BOUNDS)

    def _incl_cumsum(v):
        s = v
        for sh in (1, 2, 4, 8):
            g = _lane_gather(s, jnp.maximum(lanes - sh, 0))
            s = jnp.where(lanes >= sh, s + g, s)
        return s

    trash = lanes + _TRASH
    nf = [jnp.int32(0), jnp.int32(0)]
    lists = [idxlo, idxhi]
    for g in range(_NG):
        half = g // (_NG // 2)
        dst = lists[half]
        mv = mask_v[pl.ds(g * 16, 16)]
        rows = lanes + (g % (_NG // 2)) * 16
        m_i = jnp.where(mv != 0, 1, 0)
        incl = _incl_cumsum(m_i)
        pos_f = jnp.where(mv != 0, trash, nf[half] + lanes - incl)
        plsc.store_scatter(dst, [pos_f], rows)
        plsc.store_scatter(dst, [jnp.where(mv != 0, trash, _CELL)], rows)
        nf[half] = nf[half] + (16 - incl[15])
    for half in range(2):
        cell = jnp.full((16,), lists[half][pl.ds(_CELL, 16)][0], jnp.int32)
        lists[half][pl.ds(nf[half], 16)] = cell
    ngrp = [(nf[0] + 15) >> 4, (nf[1] + 15) >> 4]

    bufs = [bufa, bufb, bufc]
    gsems = [gsema, gsemb, gsemc]
    ssems = [ssema, ssemb, ssemc]
    z16 = jnp.zeros((16,), jnp.float32)

    def _unit_slice(u):
        c = u >> 1
        h = u & 1
        return pl.ds((c * _B + base + h * _H) * _D, _H * _D)

    def _gather(u, r):
        return pltpu.make_async_copy(
            x_hbm.at[_unit_slice(u)], bufs[r], gsems[r])

    def _scatter(u, r):
        return pltpu.make_async_copy(
            bufs[r], out_hbm.at[_unit_slice(u)], ssems[r])

    def _zero_false(half, r):
        buf = bufs[r]

        def body(g, carry):
            v = lists[half][pl.ds(g * 16, 16)] * _D
            for k in range(16):
                rowbase = v[k]
                for jj in range(_D // 16):
                    buf[pl.ds(rowbase + jj * 16, 16)] = z16
            return carry

        lax.fori_loop(0, ngrp[half], body, jnp.int32(0))

    _gather(0, 0).start()

    def qbody(q, carry):
        u0 = 6 * q
        for j in range(6):
            u = u0 + j
            r = j % 3
            rn = (j + 1) % 3

            @pl.when(u < _NU)
            def _():
                _gather(u, r).wait()
                _zero_false(j & 1, r)
                _scatter(u, r).start()

            un = u + 1

            @pl.when((un < _NU) & (un >= 3))
            def _():
                _scatter(un - 3, rn).wait()

            @pl.when(un < _NU)
            def _():
                _gather(un, rn).start()

        return carry

    lax.fori_loop(0, (_NU + 5) // 6, qbody, jnp.int32(0))
    for r in range(3):
        _scatter(0, r).wait()


_sc_call = functools.partial(
    pl.kernel,
    out_type=jax.ShapeDtypeStruct((_C * _B * _D,), jnp.float32),
    mesh=plsc.VectorSubcoreMesh(core_axis_name="c", subcore_axis_name="s"),
    compiler_params=pltpu.CompilerParams(needs_layout_passes=False),
    scratch_types=[
        pltpu.VMEM((_RPW,), jnp.int32),
        pltpu.VMEM((_IDXN,), jnp.int32),
        pltpu.VMEM((_IDXN,), jnp.int32),
        pltpu.VMEM((_H * _D,), jnp.float32),
        pltpu.VMEM((_H * _D,), jnp.float32),
        pltpu.VMEM((_H * _D,), jnp.float32),
        pltpu.SemaphoreType.DMA,
        pltpu.SemaphoreType.DMA,
        pltpu.SemaphoreType.DMA,
        pltpu.SemaphoreType.DMA,
        pltpu.SemaphoreType.DMA,
        pltpu.SemaphoreType.DMA,
        pltpu.SemaphoreType.DMA,
    ],
)(_sc_body)


def kernel(x, mask):
    x2 = jnp.transpose(x, (1, 0, 2)).reshape(_C * _B * _D)
    mask_i32 = mask.astype(jnp.int32)
    y2 = _sc_call(x2, mask_i32)
    return jnp.transpose(y2.reshape(_C, _B, _D), (1, 0, 2))


# speedup vs baseline: 2.8698x; 1.0743x over previous
import functools

import jax
import jax.numpy as jnp
from jax import lax
from jax.experimental import pallas as pl
from jax.experimental.pallas import tpu as pltpu
from jax.experimental.pallas import tpu_sc as plsc

_B, _C, _D = 16384, 26, 128
_NC, _NS = 2, 16
_NW = _NC * _NS
_RPW = _B // _NW
_NG = _RPW // 16
_K = 128
_CELL = _RPW + 16
_TRASH = _RPW + 32
_IDXN = _RPW + 48
_FULLN = _C * (_RPW + 16) + _K + 16


def _sc_body(x_hbm, mask_hbm, zrows_hbm, out_hbm,
             mask_v, idxt, idxf, fullt, fullf, bufa, bufb, bufc, zbuf,
             gsema, gsemb, gsemc, ssema, ssemb, ssemc, zsem, msem):
    wid = lax.axis_index("s") * _NC + lax.axis_index("c")
    base = wid * _RPW

    pltpu.make_async_copy(mask_hbm.at[pl.ds(base, _RPW)], mask_v, msem).start()
    pltpu.make_async_copy(zrows_hbm, zbuf, zsem).start()
    pltpu.make_async_copy(mask_hbm.at[pl.ds(base, _RPW)], mask_v, msem).wait()

    lanes = lax.iota(jnp.int32, 16)
    dnums = lax.GatherDimensionNumbers(
        offset_dims=(), collapsed_slice_dims=(0,), start_index_map=(0,))

    def _lane_gather(v, idx):
        return lax.gather(v, idx[:, None], dnums, slice_sizes=(1,),
                          mode=lax.GatherScatterMode.PROMISE_IN_BOUNDS)

    def _incl_cumsum(v):
        s = v
        for sh in (1, 2, 4, 8):
            g = _lane_gather(s, jnp.maximum(lanes - sh, 0))
            s = jnp.where(lanes >= sh, s + g, s)
        return s

    n_t = jnp.int32(0)
    n_f = jnp.int32(0)
    trash = lanes + _TRASH
    for g in range(_NG):
        mv = mask_v[pl.ds(g * 16, 16)]
        rows = lanes + (base + g * 16)
        m_i = jnp.where(mv != 0, 1, 0)
        incl = _incl_cumsum(m_i)
        pos_t = jnp.where(mv != 0, n_t + incl - 1, trash)
        pos_f = jnp.where(mv != 0, trash, n_f + lanes - incl)
        plsc.store_scatter(idxt, [pos_t], rows)
        plsc.store_scatter(idxf, [pos_f], rows)
        plsc.store_scatter(idxt, [jnp.where(mv != 0, _CELL, trash)], rows)
        plsc.store_scatter(idxf, [jnp.where(mv != 0, trash, _CELL)], rows)
        cnt = incl[15]
        n_t = n_t + cnt
        n_f = n_f + (16 - cnt)
    cell_t = jnp.full((16,), idxt[pl.ds(_CELL, 16)][0], jnp.int32)
    cell_f = jnp.full((16,), idxf[pl.ds(_CELL, 16)][0], jnp.int32)
    idxt[pl.ds(n_t, 16)] = cell_t
    idxf[pl.ds(n_f, 16)] = cell_f
    ntp = ((n_t + 15) >> 4) << 4
    nfp = ((n_f + 15) >> 4) << 4

    def _replicate(src, dst, npad):
        ngroups = npad >> 4

        def body(g, carry):
            v = src[pl.ds(g * 16, 16)]
            for c in range(_C):
                dst[pl.ds(c * npad + g * 16, 16)] = v + c * _B
            return carry

        lax.fori_loop(0, ngroups, body, jnp.int32(0))

    _replicate(idxt, fullt, ntp)
    _replicate(idxf, fullf, nfp)
    nft = _C * ntp
    nff = _C * nfp
    for j in range(_K // 16):
        fullt[pl.ds(nft + j * 16, 16)] = cell_t
        fullf[pl.ds(nff + j * 16, 16)] = cell_f

    ncht = (nft + (_K - 1)) // _K
    nchf = (nff + (_K - 1)) // _K

    pltpu.make_async_copy(zrows_hbm, zbuf, zsem).wait()
    pairs = jnp.maximum((ncht + 1) // 2, (nchf + 1) // 2)

    def pair_body(p, carry):
        c0 = 2 * p
        c1 = c0 + 1

        @pl.when(c0 < ncht)
        def _():
            pltpu.make_async_copy(
                x_hbm.at[fullt.at[pl.ds(c0 * _K, _K)]], bufa, gsema).start()

        @pl.when(c1 < ncht)
        def _():
            pltpu.make_async_copy(
                x_hbm.at[fullt.at[pl.ds(c1 * _K, _K)]], bufb, gsemb).start()

        @pl.when(c0 < nchf)
        def _():
            pltpu.make_async_copy(
                zbuf, out_hbm.at[fullf.at[pl.ds(c0 * _K, _K)]], gsemc).start()

        @pl.when(c1 < nchf)
        def _():
            pltpu.make_async_copy(
                zbuf, out_hbm.at[fullf.at[pl.ds(c1 * _K, _K)]], ssemc).start()

        @pl.when(c0 < ncht)
        def _():
            pltpu.make_async_copy(
                x_hbm.at[fullt.at[pl.ds(c0 * _K, _K)]], bufa, gsema).wait()
            pltpu.make_async_copy(
                bufa, out_hbm.at[fullt.at[pl.ds(c0 * _K, _K)]], ssema).start()

        @pl.when(c1 < ncht)
        def _():
            pltpu.make_async_copy(
                x_hbm.at[fullt.at[pl.ds(c1 * _K, _K)]], bufb, gsemb).wait()
            pltpu.make_async_copy(
                bufb, out_hbm.at[fullt.at[pl.ds(c1 * _K, _K)]], ssemb).start()

        @pl.when(c0 < ncht)
        def _():
            pltpu.make_async_copy(
                bufa, out_hbm.at[fullt.at[pl.ds(c0 * _K, _K)]], ssema).wait()

        @pl.when(c1 < ncht)
        def _():
            pltpu.make_async_copy(
                bufb, out_hbm.at[fullt.at[pl.ds(c1 * _K, _K)]], ssemb).wait()

        @pl.when(c0 < nchf)
        def _():
            pltpu.make_async_copy(
                zbuf, out_hbm.at[fullf.at[pl.ds(c0 * _K, _K)]], gsemc).wait()

        @pl.when(c1 < nchf)
        def _():
            pltpu.make_async_copy(
                zbuf, out_hbm.at[fullf.at[pl.ds(c1 * _K, _K)]], ssemc).wait()

        return carry

    lax.fori_loop(0, pairs, pair_body, jnp.int32(0))


_sc_call = functools.partial(
    pl.kernel,
    out_type=jax.ShapeDtypeStruct((_C * _B, _D), jnp.float32),
    mesh=plsc.VectorSubcoreMesh(core_axis_name="c", subcore_axis_name="s"),
    compiler_params=pltpu.CompilerParams(needs_layout_passes=False),
    scratch_types=[
        pltpu.VMEM((_RPW,), jnp.int32),
        pltpu.VMEM((_IDXN,), jnp.int32),
        pltpu.VMEM((_IDXN,), jnp.int32),
        pltpu.VMEM((_FULLN,), jnp.int32),
        pltpu.VMEM((_FULLN,), jnp.int32),
        pltpu.VMEM((_K, _D), jnp.float32),
        pltpu.VMEM((_K, _D), jnp.float32),
        pltpu.VMEM((_K, _D), jnp.float32),
        pltpu.VMEM((_K, _D), jnp.float32),
        pltpu.SemaphoreType.DMA,
        pltpu.SemaphoreType.DMA,
        pltpu.SemaphoreType.DMA,
        pltpu.SemaphoreType.DMA,
        pltpu.SemaphoreType.DMA,
        pltpu.SemaphoreType.DMA,
        pltpu.SemaphoreType.DMA,
        pltpu.SemaphoreType.DMA,
    ],
)(_sc_body)


def kernel(x, mask):
    x2 = jnp.transpose(x, (1, 0, 2)).reshape(_C * _B, _D)
    mask_i32 = mask.astype(jnp.int32)
    zrows = jnp.zeros((_K, _D), jnp.float32)
    y2 = _sc_call(x2, mask_i32, zrows)
    return jnp.transpose(y2.reshape(_C, _B, _D), (1, 0, 2))
